# Initial kernel scaffold; baseline (speedup 1.0000x reference)
import jax
import jax.numpy as jnp
from jax.experimental import pallas as pl


def kernel(x, w1, b1, g1, be1, alpha, w2, b2, g2, be2):
    raise NotImplementedError("write your pallas kernel here")


# speedup vs baseline: 2.0262x; 2.0262x over previous
import jax
import jax.numpy as jnp
from jax.experimental import pallas as pl
from jax.experimental.pallas import tpu as pltpu


def _conv_weight_pack(w_oihw, C):
    w = w_oihw.astype(jnp.float32)
    blocks = []
    for kh in range(3):
        for q in range(4):
            row = []
            for p in range(2):
                kw = q - p
                if 0 <= kw <= 2:
                    row.append(w[:, :, kh, kw].T)
                else:
                    row.append(jnp.zeros((C, C), jnp.float32))
            blocks.append(jnp.concatenate(row, axis=1))
    return jnp.concatenate(blocks, axis=0)


def _conv_pass(xin, wmat, brow, scale_row, shift_row, alpha_row, *,
               H, WP, R2, tile_r, out_dtype):
    num_tiles = R2 // tile_r
    LANES = 2 * (xin.shape[1] // 2)
    HALF = xin.shape[1] // 2

    def _body(scale_ref, shift_ref, alpha_ref, w_ref, brow_ref, x_ref,
              y_ref, stats_ref):
        f32 = jnp.float32
        x = x_ref[...].astype(f32)
        z = x * scale_ref[...] + shift_ref[...]
        a = alpha_ref[...]
        z = jnp.maximum(z, 0.0) + a * jnp.minimum(z, 0.0)
        zb = z.astype(jnp.bfloat16)

        r = jax.lax.broadcasted_iota(jnp.int32, (tile_r, 1), 0)
        j = jax.lax.rem(r, WP)
        h = jax.lax.rem(jax.lax.div(r, WP), H)
        m_top = (h > 0).astype(jnp.bfloat16)
        m_bot = (h < H - 1).astype(jnp.bfloat16)
        m_j0 = (j > 0).astype(jnp.bfloat16)
        m_j7 = (j < WP - 1).astype(jnp.bfloat16)

        zrow = jnp.zeros((WP, LANES), jnp.bfloat16)

        def half_cat(base):
            prev = jnp.concatenate([zrow[:1], base[:-1]], axis=0)[:, HALF:]
            nxt = jnp.concatenate([base[1:], zrow[:1]], axis=0)[:, :HALF]
            return jnp.concatenate([prev * m_j0, base, nxt * m_j7], axis=1)

        base_m = jnp.concatenate([zrow, zb[:-WP]], axis=0) * m_top
        base_p = jnp.concatenate([zb[WP:], zrow], axis=0) * m_bot
        lhs = jnp.concatenate(
            [half_cat(base_m), half_cat(zb), half_cat(base_p)], axis=1)

        y = jnp.dot(lhs, w_ref[...], preferred_element_type=f32)
        y = y + brow_ref[...]
        y_ref[...] = y.astype(y_ref.dtype)
        stats_ref[...] = jnp.concatenate(
            [jnp.sum(y, axis=0, keepdims=True),
             jnp.sum(y * y, axis=0, keepdims=True),
             jnp.zeros((6, LANES), f32)], axis=0)

    const2 = lambda i: (0, 0)
    return pl.pallas_call(
        _body,
        grid=(num_tiles,),
        in_specs=[
            pl.BlockSpec((1, LANES), const2),
            pl.BlockSpec((1, LANES), const2),
            pl.BlockSpec((1, LANES), const2),
            pl.BlockSpec(wmat.shape, const2),
            pl.BlockSpec((1, LANES), const2),
            pl.BlockSpec((tile_r, LANES), lambda i: (i, 0)),
        ],
        out_specs=[
            pl.BlockSpec((tile_r, LANES), lambda i: (i, 0)),
            pl.BlockSpec((8, LANES), lambda i: (i, 0)),
        ],
        out_shape=[
            jax.ShapeDtypeStruct((R2, LANES), out_dtype),
            jax.ShapeDtypeStruct((num_tiles * 8, LANES), jnp.float32),
        ],
        compiler_params=pltpu.CompilerParams(
            dimension_semantics=("parallel",),
            vmem_limit_bytes=100 << 20),
    )(scale_row, shift_row, alpha_row, wmat, brow, xin)


def _bn_residual_pass(xp, y2, scale_row, shift_row, *, R2, tile_r):
    num_tiles = R2 // tile_r
    LANES = xp.shape[1]

    def _body(scale_ref, shift_ref, x_ref, y_ref, o_ref):
        o_ref[...] = (x_ref[...]
                      + y_ref[...].astype(jnp.float32) * scale_ref[...]
                      + shift_ref[...])

    blk = pl.BlockSpec((tile_r, LANES), lambda i: (i, 0))
    return pl.pallas_call(
        _body,
        grid=(num_tiles,),
        in_specs=[pl.BlockSpec((1, LANES), lambda i: (0, 0)),
                  pl.BlockSpec((1, LANES), lambda i: (0, 0)),
                  blk, blk],
        out_specs=blk,
        out_shape=jax.ShapeDtypeStruct((R2, LANES), jnp.float32),
        compiler_params=pltpu.CompilerParams(
            dimension_semantics=("parallel",),
            vmem_limit_bytes=100 << 20),
    )(scale_row, shift_row, xp, y2)


def _bn_scale_shift(stats, gamma, beta, C, count, eps=1e-5):
    s = jnp.sum(stats.reshape(-1, 8, stats.shape[-1]), axis=0)
    ssum = jnp.sum(s[0].reshape(2, C), axis=0)
    ssq = jnp.sum(s[1].reshape(2, C), axis=0)
    mean = ssum / count
    var = jnp.maximum(ssq / count - mean * mean, 0.0)
    scale = gamma * jax.lax.rsqrt(var + eps)
    shift = beta - mean * scale
    return scale, shift


def kernel(x, w1, b1, g1, be1, alpha, w2, b2, g2, be2):
    N, C, H, W = x.shape
    WP = W // 2
    R2 = N * H * WP
    LANES = 2 * C
    count = float(N * H * W)
    f32 = jnp.float32

    rows_per_image = H * WP
    tile_r = rows_per_image
    for cand in (32, 16, 8, 4, 2, 1):
        if N % cand == 0 and cand * rows_per_image <= 4096:
            tile_r = cand * rows_per_image
            break

    xp = jnp.transpose(x, (0, 2, 3, 1)).reshape(R2, LANES).astype(f32)
    xb = xp.astype(jnp.bfloat16)

    wm1 = _conv_weight_pack(w1, C).astype(jnp.bfloat16)
    wm2 = _conv_weight_pack(w2, C).astype(jnp.bfloat16)

    def lane_row(per_channel):
        return jnp.tile(per_channel.astype(f32), 2).reshape(1, LANES)

    ones_row = lane_row(jnp.ones((C,), f32))
    zeros_row = jnp.zeros((1, LANES), f32)
    alpha_row = lane_row(alpha.reshape(()) * jnp.ones((C,), f32))

    kw = dict(H=H, WP=WP, R2=R2, tile_r=tile_r, out_dtype=jnp.bfloat16)

    y1, st1 = _conv_pass(xb, wm1, lane_row(b1), ones_row, zeros_row,
                         ones_row, **kw)
    sc1, sh1 = _bn_scale_shift(st1, g1, be1, C, count)

    y2, st2 = _conv_pass(y1, wm2, lane_row(b2), lane_row(sc1),
                         lane_row(sh1), alpha_row, **kw)
    sc2, sh2 = _bn_scale_shift(st2, g2, be2, C, count)

    out_flat = _bn_residual_pass(xp, y2, lane_row(sc2), lane_row(sh2),
                                 R2=R2, tile_r=tile_r)

    out = out_flat.reshape(N, H, W, C)
    return jnp.transpose(out, (0, 3, 1, 2))
